# baseline (speedup 1.0000x reference)
import jax
import jax.numpy as jnp
from jax import lax
from jax.experimental import pallas as pl
from jax.experimental.pallas import tpu as pltpu
from jax.experimental.pallas import tpu_sc as plsc

NUM_ENTITIES = 1000000
HALF = 32
EMBED = 64
ROWS = 16384
FEATS = 26
NC = 2
NS = 16
NW = NC * NS
BPW = ROWS // NW
CHUNK = 128
NBUF = 4


def _body(idx_hbm, bra_hbm, ket_hbm, bra_out, ket_out,
          idx_v, bg_v, kg_v, bstg, kstg, gsems, wsems):
    wid = lax.axis_index("s") * NC + lax.axis_index("c")
    b0 = wid * BPW

    zeros16 = jnp.zeros((16,), jnp.float32)

    pltpu.sync_copy(idx_hbm.at[:, pl.ds(b0, BPW)], idx_v)

    @plsc.parallel_loop(0, CHUNK, unroll=4)
    def zrow(r):
        for s in range(NBUF):
            bstg[s, r, pl.ds(HALF, 16)] = zeros16
            bstg[s, r, pl.ds(HALF + 16, 16)] = zeros16
            kstg[s, r, pl.ds(0, 16)] = zeros16
            kstg[s, r, pl.ds(16, 16)] = zeros16

    def drain(s, f):
        row0 = b0 + s * CHUNK
        pltpu.make_async_copy(
            bstg.at[s], bra_out.at[pl.ds(row0, CHUNK), f], wsems.at[s]
        ).wait()
        pltpu.make_async_copy(
            kstg.at[s], ket_out.at[pl.ds(row0, CHUNK), f], wsems.at[s]
        ).wait()

    def group(f, c):
        for s in range(NBUF):
            @pl.when(f >= 1)
            def _(s=s):
                drain(s, f - 1)

            idx_ref = idx_v.at[f, pl.ds(s * CHUNK, CHUNK)]
            pltpu.async_copy(bra_hbm.at[idx_ref], bg_v.at[s], gsems.at[s])
            pltpu.async_copy(ket_hbm.at[idx_ref], kg_v.at[s], gsems.at[s])

        for s in range(NBUF):
            idx_ref = idx_v.at[f, pl.ds(s * CHUNK, CHUNK)]
            pltpu.make_async_copy(
                bra_hbm.at[idx_ref], bg_v.at[s], gsems.at[s]).wait()
            pltpu.make_async_copy(
                ket_hbm.at[idx_ref], kg_v.at[s], gsems.at[s]).wait()

            @plsc.parallel_loop(0, CHUNK, unroll=4)
            def crow(r, s=s):
                bstg[s, r, pl.ds(0, 16)] = bg_v[s, r, pl.ds(0, 16)]
                bstg[s, r, pl.ds(16, 16)] = bg_v[s, r, pl.ds(16, 16)]
                kstg[s, r, pl.ds(HALF, 16)] = kg_v[s, r, pl.ds(0, 16)]
                kstg[s, r, pl.ds(HALF + 16, 16)] = kg_v[s, r, pl.ds(16, 16)]

            row0 = b0 + s * CHUNK
            pltpu.async_copy(bstg.at[s], bra_out.at[pl.ds(row0, CHUNK), f],
                             wsems.at[s])
            pltpu.async_copy(kstg.at[s], ket_out.at[pl.ds(row0, CHUNK), f],
                             wsems.at[s])
        return c

    lax.fori_loop(0, FEATS, group, 0)

    for s in range(NBUF):
        drain(s, FEATS - 1)


@jax.jit
def _run(idx_t, bra_w, ket_w):
    mesh = plsc.VectorSubcoreMesh(core_axis_name="c", subcore_axis_name="s")
    out = pl.kernel(
        _body,
        out_type=(
            jax.ShapeDtypeStruct((ROWS, FEATS, EMBED), jnp.float32),
            jax.ShapeDtypeStruct((ROWS, FEATS, EMBED), jnp.float32),
        ),
        mesh=mesh,
        compiler_params=pltpu.CompilerParams(use_tc_tiling_on_sc=False,
                                             needs_layout_passes=False),
        scratch_types=[
            pltpu.VMEM((FEATS, BPW), jnp.int32),
            pltpu.VMEM((NBUF, CHUNK, HALF), jnp.float32),
            pltpu.VMEM((NBUF, CHUNK, HALF), jnp.float32),
            pltpu.VMEM((NBUF, CHUNK, EMBED), jnp.float32),
            pltpu.VMEM((NBUF, CHUNK, EMBED), jnp.float32),
            pltpu.SemaphoreType.DMA((NBUF,)),
            pltpu.SemaphoreType.DMA((NBUF,)),
        ],
    )(idx_t, bra_w, ket_w)
    return out


def kernel(index, bra_w, ket_w):
    idx_t = jnp.transpose(index.astype(jnp.int32))
    return _run(idx_t, bra_w, ket_w)


# speedup vs baseline: 1.2692x; 1.2692x over previous
import jax
import jax.numpy as jnp
from jax import lax
from jax.experimental import pallas as pl
from jax.experimental.pallas import tpu as pltpu
from jax.experimental.pallas import tpu_sc as plsc

NUM_ENTITIES = 1000000
HALF = 32
EMBED = 64
ROWS = 16384
FEATS = 26
NC = 2
NS = 16
NW = NC * NS
BPW = ROWS // NW
CHUNK = 128
NBUF = 4


def _body(idx_hbm, bra_hbm, ket_hbm, bra_out, ket_out,
          idx_v, bg_v, kg_v, bstg, kstg, gsems, wsems):
    wid = lax.axis_index("s") * NC + lax.axis_index("c")
    b0 = wid * BPW

    zeros16 = jnp.zeros((16,), jnp.float32)

    pltpu.sync_copy(idx_hbm.at[:, pl.ds(b0, BPW)], idx_v)

    @plsc.parallel_loop(0, CHUNK, unroll=4)
    def zrow(r):
        for s in range(NBUF):
            bstg[s, r, pl.ds(HALF, 16)] = zeros16
            bstg[s, r, pl.ds(HALF + 16, 16)] = zeros16
            kstg[s, r, pl.ds(0, 16)] = zeros16
            kstg[s, r, pl.ds(16, 16)] = zeros16

    def drain(s, f):
        row0 = b0 + s * CHUNK
        pltpu.make_async_copy(
            bstg.at[s], bra_out.at[pl.ds(row0, CHUNK), f // 8, f % 8,
                                   pl.ds(0, EMBED)], wsems.at[s]
        ).wait()
        pltpu.make_async_copy(
            kstg.at[s], ket_out.at[pl.ds(row0, CHUNK), f // 8, f % 8,
                                   pl.ds(0, EMBED)], wsems.at[s]
        ).wait()

    def group(f, c):
        for s in range(NBUF):
            @pl.when(f >= 1)
            def _(s=s):
                drain(s, f - 1)

            idx_ref = idx_v.at[f, pl.ds(s * CHUNK, CHUNK)]
            pltpu.async_copy(bra_hbm.at[idx_ref], bg_v.at[s], gsems.at[s])
            pltpu.async_copy(ket_hbm.at[idx_ref], kg_v.at[s], gsems.at[s])

        for s in range(NBUF):
            idx_ref = idx_v.at[f, pl.ds(s * CHUNK, CHUNK)]
            pltpu.make_async_copy(
                bra_hbm.at[idx_ref], bg_v.at[s], gsems.at[s]).wait()
            pltpu.make_async_copy(
                ket_hbm.at[idx_ref], kg_v.at[s], gsems.at[s]).wait()

            @plsc.parallel_loop(0, CHUNK, unroll=4)
            def crow(r, s=s):
                bstg[s, r, pl.ds(0, 16)] = bg_v[s, r, pl.ds(0, 16)]
                bstg[s, r, pl.ds(16, 16)] = bg_v[s, r, pl.ds(16, 16)]
                kstg[s, r, pl.ds(HALF, 16)] = kg_v[s, r, pl.ds(0, 16)]
                kstg[s, r, pl.ds(HALF + 16, 16)] = kg_v[s, r, pl.ds(16, 16)]

            row0 = b0 + s * CHUNK
            pltpu.async_copy(
                bstg.at[s], bra_out.at[pl.ds(row0, CHUNK), f // 8, f % 8,
                                       pl.ds(0, EMBED)], wsems.at[s])
            pltpu.async_copy(
                kstg.at[s], ket_out.at[pl.ds(row0, CHUNK), f // 8, f % 8,
                                       pl.ds(0, EMBED)], wsems.at[s])
        return c

    lax.fori_loop(0, FEATS, group, 0)

    for s in range(NBUF):
        drain(s, FEATS - 1)


@jax.jit
def _run(idx_t, bra_w, ket_w):
    mesh = plsc.VectorSubcoreMesh(core_axis_name="c", subcore_axis_name="s")
    out = pl.kernel(
        _body,
        out_type=(
            jax.ShapeDtypeStruct((ROWS, 4, 8, 128), jnp.float32),
            jax.ShapeDtypeStruct((ROWS, 4, 8, 128), jnp.float32),
        ),
        mesh=mesh,
        compiler_params=pltpu.CompilerParams(use_tc_tiling_on_sc=False,
                                             needs_layout_passes=False),
        scratch_types=[
            pltpu.VMEM((FEATS, BPW), jnp.int32),
            pltpu.VMEM((NBUF, CHUNK, HALF), jnp.float32),
            pltpu.VMEM((NBUF, CHUNK, HALF), jnp.float32),
            pltpu.VMEM((NBUF, CHUNK, EMBED), jnp.float32),
            pltpu.VMEM((NBUF, CHUNK, EMBED), jnp.float32),
            pltpu.SemaphoreType.DMA((NBUF,)),
            pltpu.SemaphoreType.DMA((NBUF,)),
        ],
    )(idx_t, bra_w, ket_w)
    return out


def _view(p):
    return p.reshape(ROWS, 32, 128)[:, :FEATS, :EMBED]


def kernel(index, bra_w, ket_w):
    idx_t = jnp.transpose(index.astype(jnp.int32))
    p_bra, p_ket = _run(idx_t, bra_w, ket_w)
    return (_view(p_bra), _view(p_ket))


# speedup vs baseline: 1.3280x; 1.0463x over previous
import jax
import jax.numpy as jnp
from jax import lax
from jax.experimental import pallas as pl
from jax.experimental.pallas import tpu as pltpu
from jax.experimental.pallas import tpu_sc as plsc

NUM_ENTITIES = 1000000
HALF = 32
EMBED = 64
ROWS = 16384
FEATS = 26
NC = 2
NS = 16
NW = NC * NS
BPW = ROWS // NW
CHUNK = 128
NBUF = 4
EB = EMBED // 8
HB = HALF // 8
BB = ROWS // CHUNK


def _body(idx_hbm, bra_hbm, ket_hbm, bra_out, ket_out,
          idx_v, bg_v, kg_v, bstg, kstg, gsems, wsems):
    wid = lax.axis_index("s") * NC + lax.axis_index("c")
    b0 = wid * BPW

    zeros16 = jnp.zeros((16,), jnp.float32)
    iota16 = lax.iota(jnp.int32, 16)
    colmod = [(iota16 + k) & 15 for k in range(16)]

    pltpu.sync_copy(idx_hbm.at[:, pl.ds(b0, BPW)], idx_v)

    @plsc.parallel_loop(0, HALF, unroll=4)
    def zrow(r):
        for s in range(NBUF):
            for h in range(CHUNK // 16):
                bstg[s, HB + r // 8, r % 8, pl.ds(h * 16, 16)] = zeros16
                kstg[s, r // 8, r % 8, pl.ds(h * 16, 16)] = zeros16

    def drain(s, f):
        bblk = wid * NBUF + s
        pltpu.make_async_copy(
            bstg.at[s], bra_out.at[f, slice(None), bblk], wsems.at[s]
        ).wait()
        pltpu.make_async_copy(
            kstg.at[s], ket_out.at[f, slice(None), bblk], wsems.at[s]
        ).wait()

    def group(f, c):
        for s in range(NBUF):
            @pl.when(f >= 1)
            def _(s=s):
                drain(s, f - 1)

            idx_ref = idx_v.at[f, pl.ds(s * CHUNK, CHUNK)]
            pltpu.async_copy(bra_hbm.at[idx_ref], bg_v.at[s], gsems.at[s])
            pltpu.async_copy(ket_hbm.at[idx_ref], kg_v.at[s], gsems.at[s])

        for s in range(NBUF):
            idx_ref = idx_v.at[f, pl.ds(s * CHUNK, CHUNK)]
            pltpu.make_async_copy(
                bra_hbm.at[idx_ref], bg_v.at[s], gsems.at[s]).wait()
            pltpu.make_async_copy(
                ket_hbm.at[idx_ref], kg_v.at[s], gsems.at[s]).wait()

            @plsc.parallel_loop(0, CHUNK // 16, unroll=2)
            def tloop(rb, s=s):
                rows = iota16 + rb * 16
                for cb in range(HALF // 16):
                    for k in range(16):
                        cols = colmod[k] + cb * 16
                        eb = lax.shift_right_logical(cols, 3)
                        es = cols & 7
                        vb = plsc.load_gather(bg_v.at[s], [rows, cols])
                        plsc.store_scatter(bstg.at[s], [eb, es, rows], vb)
                        vk = plsc.load_gather(kg_v.at[s], [rows, cols])
                        plsc.store_scatter(kstg.at[s], [eb + HB, es, rows], vk)

            bblk = wid * NBUF + s
            pltpu.async_copy(bstg.at[s], bra_out.at[f, slice(None), bblk],
                             wsems.at[s])
            pltpu.async_copy(kstg.at[s], ket_out.at[f, slice(None), bblk],
                             wsems.at[s])
        return c

    lax.fori_loop(0, FEATS, group, 0)

    for s in range(NBUF):
        drain(s, FEATS - 1)


@jax.jit
def _run(idx_t, bra_w, ket_w):
    mesh = plsc.VectorSubcoreMesh(core_axis_name="c", subcore_axis_name="s")
    out = pl.kernel(
        _body,
        out_type=(
            jax.ShapeDtypeStruct((FEATS, EB, BB, 8, CHUNK), jnp.float32),
            jax.ShapeDtypeStruct((FEATS, EB, BB, 8, CHUNK), jnp.float32),
        ),
        mesh=mesh,
        compiler_params=pltpu.CompilerParams(use_tc_tiling_on_sc=False,
                                             needs_layout_passes=False),
        scratch_types=[
            pltpu.VMEM((FEATS, BPW), jnp.int32),
            pltpu.VMEM((NBUF, CHUNK, HALF), jnp.float32),
            pltpu.VMEM((NBUF, CHUNK, HALF), jnp.float32),
            pltpu.VMEM((NBUF, EB, 8, CHUNK), jnp.float32),
            pltpu.VMEM((NBUF, EB, 8, CHUNK), jnp.float32),
            pltpu.SemaphoreType.DMA((NBUF,)),
            pltpu.SemaphoreType.DMA((NBUF,)),
        ],
    )(idx_t, bra_w, ket_w)
    return out


def _detile(p):
    return p.transpose(2, 4, 0, 1, 3).reshape(ROWS, FEATS, EMBED)


def kernel(index, bra_w, ket_w):
    idx_t = jnp.transpose(index.astype(jnp.int32))
    p_bra, p_ket = _run(idx_t, bra_w, ket_w)
    return (_detile(p_bra), _detile(p_ket))
